# Initial kernel scaffold; baseline (speedup 1.0000x reference)
import functools

import jax
import jax.numpy as jnp
from jax import lax
from jax.experimental import pallas as pl
from jax.experimental.pallas import tpu as pltpu
from jax.experimental.pallas import tpu_sc as plsc

B = 1024
D = 64
H = 128
L = 20

NC = 2
NS = 16
NW = NC * NS
B_PER_W = B // NW
IDX_PER_W = B_PER_W * L
IDX_CHUNK = 128
N_CHUNKS = IDX_PER_W // IDX_CHUNK
D_VECS = D // 16


def _gather_mean_body(idx_hbm, table_hbm, out_hbm, idx_v, rows_v, out_v, sem):
    wid = lax.axis_index("s") * NC + lax.axis_index("c")

    pltpu.sync_copy(idx_hbm.at[pl.ds(wid * N_CHUNKS, N_CHUNKS)], idx_v)

    copies = []
    for j in range(N_CHUNKS):
        copies.append(
            pltpu.async_copy(
                table_hbm.at[idx_v.at[j]],
                rows_v.at[pl.ds(j * IDX_CHUNK, IDX_CHUNK)],
                sem,
            )
        )
    for c in copies:
        c.wait()

    inv_l = 1.0 / L

    def body(b, carry):
        base = b * L
        for g in range(D_VECS):
            acc = jnp.zeros((16,), jnp.float32)
            for l in range(L):
                acc = acc + rows_v[base + l, pl.ds(g * 16, 16)]
            out_v[b, pl.ds(g * 16, 16)] = acc * inv_l
        return carry

    lax.fori_loop(0, B_PER_W, body, 0)

    pltpu.sync_copy(out_v, out_hbm.at[pl.ds(wid * B_PER_W, B_PER_W)])


_gather_mean = functools.partial(
    pl.kernel,
    out_type=jax.ShapeDtypeStruct((B, D), jnp.float32),
    mesh=plsc.VectorSubcoreMesh(core_axis_name="c", subcore_axis_name="s"),
    scratch_types=[
        pltpu.VMEM((N_CHUNKS, IDX_CHUNK), jnp.int32),
        pltpu.VMEM((IDX_PER_W, D), jnp.float32),
        pltpu.VMEM((B_PER_W, D), jnp.float32),
        pltpu.SemaphoreType.DMA,
    ],
)(_gather_mean_body)


def _dense_body(x_ref, w_ih_ref, b_ih_ref, b_hh_ref, w_out_ref, b_out_ref,
                out_ref):
    x = x_ref[...]
    gi = lax.dot_general(
        x, w_ih_ref[...], (((1,), (1,)), ((), ())),
        preferred_element_type=jnp.float32,
    ) + b_ih_ref[...]
    hb = b_hh_ref[...]
    r = jax.nn.sigmoid(gi[:, :H] + hb[:, :H])
    z = jax.nn.sigmoid(gi[:, H:2 * H] + hb[:, H:2 * H])
    n = jnp.tanh(gi[:, 2 * H:] + r * hb[:, 2 * H:])
    h = (1.0 - z) * n
    logit = jnp.sum(h * w_out_ref[...], axis=1, keepdims=True) + b_out_ref[...]
    out_ref[...] = jax.nn.sigmoid(logit)


def _dense(x, w_ih, b_ih2, b_hh2, w_out, b_out2):
    return pl.pallas_call(
        _dense_body,
        out_shape=jax.ShapeDtypeStruct((B, 1), jnp.float32),
        in_specs=[pl.BlockSpec(memory_space=pltpu.VMEM)] * 6,
        out_specs=pl.BlockSpec(memory_space=pltpu.VMEM),
    )(x, w_ih, b_ih2, b_hh2, w_out, b_out2)


def kernel(indices, labels, table, w_ih, w_hh, b_ih, b_hh, w_out, b_out):
    del w_hh
    idx0 = indices[:, 0, :].reshape(NW * N_CHUNKS, IDX_CHUNK)
    x0 = _gather_mean(idx0, table)
    pred = _dense(
        x0,
        w_ih,
        b_ih.reshape(1, 3 * H),
        b_hh.reshape(1, 3 * H),
        w_out,
        b_out.reshape(1, 1),
    )
    return (pred, labels)


# speedup vs baseline: 2.8206x; 2.8206x over previous
import functools

import jax
import jax.numpy as jnp
from jax import lax
from jax.experimental import pallas as pl
from jax.experimental.pallas import tpu as pltpu
from jax.experimental.pallas import tpu_sc as plsc

B = 1024
D = 64
H = 128
L = 20

NC = 2
NS = 16
NW = NC * NS
B_PER_W = B // NW
IDX_PER_W = B_PER_W * L
IDX_CHUNK = 128
N_CHUNKS = IDX_PER_W // IDX_CHUNK
D_VECS = D // 16


def _gather_mean_body(idx_hbm, table_hbm, out_hbm, idx_v, rows_v, out_v, sem):
    wid = lax.axis_index("s") * NC + lax.axis_index("c")

    pltpu.sync_copy(idx_hbm.at[wid], idx_v)

    copies = []
    for j in range(N_CHUNKS):
        copies.append(
            pltpu.async_copy(
                table_hbm.at[idx_v.at[j]],
                rows_v.at[pl.ds(j * IDX_CHUNK, IDX_CHUNK)],
                sem,
            )
        )
    for c in copies:
        c.wait()

    inv_l = 1.0 / L

    def body(b, carry):
        base = b * L
        for g in range(D_VECS):
            acc = jnp.zeros((16,), jnp.float32)
            for l in range(L):
                acc = acc + rows_v[base + l, pl.ds(g * 16, 16)]
            out_v[b, pl.ds(g * 16, 16)] = acc * inv_l
        return carry

    lax.fori_loop(0, B_PER_W, body, 0)

    pltpu.sync_copy(out_v, out_hbm.at[pl.ds(wid * B_PER_W, B_PER_W)])


@functools.cache
def _gather_mean():
    return pl.kernel(
        _gather_mean_body,
        out_type=jax.ShapeDtypeStruct((B, D), jnp.float32),
        mesh=plsc.VectorSubcoreMesh(core_axis_name="c", subcore_axis_name="s"),
        scratch_types=[
            pltpu.VMEM((N_CHUNKS, IDX_CHUNK), jnp.int32),
            pltpu.VMEM((IDX_PER_W, D), jnp.float32),
            pltpu.VMEM((B_PER_W, D), jnp.float32),
            pltpu.SemaphoreType.DMA,
        ],
        compiler_params=pltpu.CompilerParams(use_tc_tiling_on_sc=False),
    )


def _dense_body(x_ref, w_ih_ref, b_ih_ref, b_hh_ref, w_out_ref, b_out_ref,
                out_ref):
    x = x_ref[...]
    gi = lax.dot_general(
        x, w_ih_ref[...], (((1,), (1,)), ((), ())),
        preferred_element_type=jnp.float32,
    ) + b_ih_ref[...]
    hb = b_hh_ref[...]
    r = jax.nn.sigmoid(gi[:, :H] + hb[:, :H])
    z = jax.nn.sigmoid(gi[:, H:2 * H] + hb[:, H:2 * H])
    n = jnp.tanh(gi[:, 2 * H:] + r * hb[:, 2 * H:])
    h = (1.0 - z) * n
    logit = jnp.sum(h * w_out_ref[...], axis=1, keepdims=True) + b_out_ref[...]
    out_ref[...] = jax.nn.sigmoid(logit)


def _dense(x, w_ih, b_ih2, b_hh2, w_out, b_out2):
    return pl.pallas_call(
        _dense_body,
        out_shape=jax.ShapeDtypeStruct((B, 1), jnp.float32),
        in_specs=[pl.BlockSpec(memory_space=pltpu.VMEM)] * 6,
        out_specs=pl.BlockSpec(memory_space=pltpu.VMEM),
    )(x, w_ih, b_ih2, b_hh2, w_out, b_out2)


def kernel(indices, labels, table, w_ih, w_hh, b_ih, b_hh, w_out, b_out):
    del w_hh
    idx0 = indices[:, 0, :].reshape(NW, N_CHUNKS, IDX_CHUNK)
    x0 = _gather_mean()(idx0, table)
    pred = _dense(
        x0,
        w_ih,
        b_ih.reshape(1, 3 * H),
        b_hh.reshape(1, 3 * H),
        w_out,
        b_out.reshape(1, 1),
    )
    return (pred, labels)
